# Initial kernel scaffold; baseline (speedup 1.0000x reference)
import jax
import jax.numpy as jnp
from jax.experimental import pallas as pl


def kernel(boxes, scores):
    raise NotImplementedError("write your pallas kernel here")


# speedup vs baseline: 14.4773x; 14.4773x over previous
import jax
import jax.numpy as jnp
from jax.experimental import pallas as pl
from jax.experimental.pallas import tpu as pltpu

_N = 20000
_K = 1000
_D = 100
_ROWS = 160
_PADN = _ROWS * 128
_NMS_T = 0.5
_SCORE_T = 0.05
_BIG = 2 ** 30


def _rmax(a):
    t = jnp.max(a, axis=0, keepdims=True)
    return jnp.max(t, axis=1, keepdims=True)


def _rmin(a):
    t = jnp.min(a, axis=0, keepdims=True)
    return jnp.min(t, axis=1, keepdims=True)


def _krn(bx1, by1, bx2, by2, sc,
         ox1, oy1, ox2, oy2, osc,
         msk, idxp, jsel, sx1, sy1, sx2, sy2, ssc, sar, keep, fin):
    ridx = jax.lax.broadcasted_iota(jnp.int32, (_ROWS, 128), 0)
    cidx = jax.lax.broadcasted_iota(jnp.int32, (_ROWS, 128), 1)
    flat = ridx * 128 + cidx
    idxp[...] = flat
    valid = flat < _N
    s = sc[...]
    msk[...] = jnp.where(valid & (s > _SCORE_T), s,
                         jnp.where(valid, -1.0, -2.0))
    flatk = (jax.lax.broadcasted_iota(jnp.int32, (8, 128), 0) * 128
             + jax.lax.broadcasted_iota(jnp.int32, (8, 128), 1))

    ssc[...] = jnp.full((8, 128), -2.0, jnp.float32)
    jsel[...] = jnp.zeros((8, 128), jnp.int32)
    sx1[...] = jnp.zeros((8, 128), jnp.float32)
    sy1[...] = jnp.zeros((8, 128), jnp.float32)
    sx2[...] = jnp.zeros((8, 128), jnp.float32)
    sy2[...] = jnp.zeros((8, 128), jnp.float32)
    keep[...] = jnp.ones((8, 128), jnp.float32)
    ox1[...] = jnp.zeros((8, 128), jnp.float32)
    oy1[...] = jnp.zeros((8, 128), jnp.float32)
    ox2[...] = jnp.zeros((8, 128), jnp.float32)
    oy2[...] = jnp.zeros((8, 128), jnp.float32)
    osc[...] = jnp.zeros((8, 128), jnp.float32)

    def sel_body(i, carry):
        a = msk[...]
        m11 = _rmax(a)
        j11 = _rmin(jnp.where(a == m11, idxp[...], _BIG))
        msk[...] = jnp.where(idxp[...] == j11, -3.0, a)
        sloti = flatk == i
        ssc[...] = jnp.where(sloti, jnp.broadcast_to(m11, (8, 128)),
                             ssc[...])
        jsel[...] = jnp.where(sloti, jnp.broadcast_to(j11, (8, 128)),
                              jsel[...])
        return carry

    jax.lax.fori_loop(0, _K, sel_body, 0, unroll=4)

    def gat_body(i, carry):
        sloti = flatk == i
        j11 = _rmax(jnp.where(sloti, jsel[...], 0))
        oh = idxp[...] == j11

        def ext(ref):
            return _rmax(jnp.where(oh, ref[...], 0.0))

        x1 = ext(bx1)
        y1 = ext(by1)
        x2 = ext(bx2)
        y2 = ext(by2)
        b8 = lambda v: jnp.broadcast_to(v, (8, 128))
        sx1[...] = jnp.where(sloti, b8(x1), sx1[...])
        sy1[...] = jnp.where(sloti, b8(y1), sy1[...])
        sx2[...] = jnp.where(sloti, b8(x2), sx2[...])
        sy2[...] = jnp.where(sloti, b8(y2), sy2[...])
        return carry

    jax.lax.fori_loop(0, _K, gat_body, 0, unroll=4)

    sar[...] = (jnp.maximum(sx2[...] - sx1[...], 0.0)
                * jnp.maximum(sy2[...] - sy1[...], 0.0))

    vx1 = sx1[...]
    vy1 = sy1[...]
    vx2 = sx2[...]
    vy2 = sy2[...]
    var = sar[...]
    vsc = ssc[...]

    def nms_body(i, carry):
        sloti = flatk == i
        kv = keep[...]

        def bext(plane):
            return _rmax(jnp.where(sloti, plane, -2e9))

        x1i = bext(vx1)
        y1i = bext(vy1)
        x2i = bext(vx2)
        y2i = bext(vy2)
        ai = bext(var)
        si = bext(vsc)
        ki = bext(kv)
        xx1 = jnp.maximum(x1i, vx1)
        yy1 = jnp.maximum(y1i, vy1)
        xx2 = jnp.minimum(x2i, vx2)
        yy2 = jnp.minimum(y2i, vy2)
        inter = jnp.maximum(xx2 - xx1, 0.0) * jnp.maximum(yy2 - yy1, 0.0)
        iou = inter / (ai + var - inter + 1e-9)
        sup = ((iou > _NMS_T) & (flatk > i)
               & (ki > 0.5) & (si > 0.0))
        keep[...] = jnp.where(sup, 0.0, kv)
        return carry

    jax.lax.fori_loop(0, _K, nms_body, 0, unroll=4)

    kv = keep[...]
    validk = flatk < _K
    fin[...] = jnp.where(validk,
                         jnp.where((kv > 0.5) & (vsc > 0.0), vsc, -1.0),
                         -1e9)

    def top_body(d, carry):
        a = fin[...]
        m11 = _rmax(a)
        j11 = _rmin(jnp.where(a == m11, flatk, _BIG))
        oh = flatk == j11
        fin[...] = jnp.where(oh, -2e9, a)

        def bext(plane):
            return _rmax(jnp.where(oh, plane, -2e9))

        b8 = lambda v: jnp.broadcast_to(v, (8, 128))
        slotd = flatk == d
        ox1[...] = jnp.where(slotd, b8(bext(vx1)), ox1[...])
        oy1[...] = jnp.where(slotd, b8(bext(vy1)), oy1[...])
        ox2[...] = jnp.where(slotd, b8(bext(vx2)), ox2[...])
        oy2[...] = jnp.where(slotd, b8(bext(vy2)), oy2[...])
        osc[...] = jnp.where(slotd, b8(m11), osc[...])
        return carry

    jax.lax.fori_loop(0, _D, top_body, 0, unroll=4)


def _pad_plane(v, pad_val):
    p = jnp.concatenate(
        [v, jnp.full((_PADN - _N,), pad_val, v.dtype)])
    return p.reshape(_ROWS, 128)


def kernel(boxes, scores):
    boxes = boxes.astype(jnp.float32)
    scores = scores.astype(jnp.float32)
    bx1 = _pad_plane(boxes[:, 0], 0.0)
    by1 = _pad_plane(boxes[:, 1], 0.0)
    bx2 = _pad_plane(boxes[:, 2], 0.0)
    by2 = _pad_plane(boxes[:, 3], 0.0)
    sc = _pad_plane(scores, 0.0)
    shp = jax.ShapeDtypeStruct((8, 128), jnp.float32)
    ox1, oy1, ox2, oy2, osc = pl.pallas_call(
        _krn,
        out_shape=(shp, shp, shp, shp, shp),
        scratch_shapes=[
            pltpu.VMEM((_ROWS, 128), jnp.float32),
            pltpu.VMEM((_ROWS, 128), jnp.int32),
            pltpu.VMEM((8, 128), jnp.int32),
            pltpu.VMEM((8, 128), jnp.float32),
            pltpu.VMEM((8, 128), jnp.float32),
            pltpu.VMEM((8, 128), jnp.float32),
            pltpu.VMEM((8, 128), jnp.float32),
            pltpu.VMEM((8, 128), jnp.float32),
            pltpu.VMEM((8, 128), jnp.float32),
            pltpu.VMEM((8, 128), jnp.float32),
            pltpu.VMEM((8, 128), jnp.float32),
        ],
    )(bx1, by1, bx2, by2, sc)
    det = jnp.stack([ox1.reshape(-1)[:_D], oy1.reshape(-1)[:_D],
                     ox2.reshape(-1)[:_D], oy2.reshape(-1)[:_D],
                     osc.reshape(-1)[:_D]], axis=1)
    return det


# speedup vs baseline: 14.5627x; 1.0059x over previous
import jax
import jax.numpy as jnp
from jax.experimental import pallas as pl
from jax.experimental.pallas import tpu as pltpu

_N = 20000
_K = 1000
_D = 100
_ROWS = 160
_PADN = _ROWS * 128
_NMS_T = 0.5
_SCORE_T = 0.05
_BIG = 2 ** 30


def _rmax(a):
    t = jnp.max(a, axis=0, keepdims=True)
    return jnp.max(t, axis=1, keepdims=True)


def _rmin(a):
    t = jnp.min(a, axis=0, keepdims=True)
    return jnp.min(t, axis=1, keepdims=True)


def _krn(bx1, by1, bx2, by2, sc,
         ox1, oy1, ox2, oy2, osc,
         jsel, sx1, sy1, sx2, sy2, ssc):
    ridx = jax.lax.broadcasted_iota(jnp.int32, (_ROWS, 128), 0)
    cidx = jax.lax.broadcasted_iota(jnp.int32, (_ROWS, 128), 1)
    flat = ridx * 128 + cidx
    valid = flat < _N
    s = sc[...]
    a0 = jnp.where(valid & (s > _SCORE_T), s,
                   jnp.where(valid, -1.0, -2.0))
    flatk = (jax.lax.broadcasted_iota(jnp.int32, (8, 128), 0) * 128
             + jax.lax.broadcasted_iota(jnp.int32, (8, 128), 1))

    ssc[...] = jnp.full((8, 128), -2.0, jnp.float32)
    jsel[...] = jnp.zeros((8, 128), jnp.int32)
    sx1[...] = jnp.zeros((8, 128), jnp.float32)
    sy1[...] = jnp.zeros((8, 128), jnp.float32)
    sx2[...] = jnp.zeros((8, 128), jnp.float32)
    sy2[...] = jnp.zeros((8, 128), jnp.float32)
    ox1[...] = jnp.zeros((8, 128), jnp.float32)
    oy1[...] = jnp.zeros((8, 128), jnp.float32)
    ox2[...] = jnp.zeros((8, 128), jnp.float32)
    oy2[...] = jnp.zeros((8, 128), jnp.float32)
    osc[...] = jnp.zeros((8, 128), jnp.float32)

    def sel_body(i, a):
        m11 = _rmax(a)
        j11 = _rmin(jnp.where(a == m11, flat, _BIG))
        sloti = flatk == i
        ssc[...] = jnp.where(sloti, jnp.broadcast_to(m11, (8, 128)),
                             ssc[...])
        jsel[...] = jnp.where(sloti, jnp.broadcast_to(j11, (8, 128)),
                              jsel[...])
        return jnp.where(flat == j11, -3.0, a)

    jax.lax.fori_loop(0, _K, sel_body, a0, unroll=4)

    def gat_body(i, carry):
        sloti = flatk == i
        j11 = _rmax(jnp.where(sloti, jsel[...], 0))
        oh = flat == j11

        def ext(ref):
            return _rmax(jnp.where(oh, ref[...], 0.0))

        x1 = ext(bx1)
        y1 = ext(by1)
        x2 = ext(bx2)
        y2 = ext(by2)
        b8 = lambda v: jnp.broadcast_to(v, (8, 128))
        sx1[...] = jnp.where(sloti, b8(x1), sx1[...])
        sy1[...] = jnp.where(sloti, b8(y1), sy1[...])
        sx2[...] = jnp.where(sloti, b8(x2), sx2[...])
        sy2[...] = jnp.where(sloti, b8(y2), sy2[...])
        return carry

    jax.lax.fori_loop(0, _K, gat_body, 0, unroll=4)

    vx1 = sx1[...]
    vy1 = sy1[...]
    vx2 = sx2[...]
    vy2 = sy2[...]
    var = (jnp.maximum(vx2 - vx1, 0.0)
           * jnp.maximum(vy2 - vy1, 0.0))
    vsc = ssc[...]

    def nms_body(i, kv):
        sloti = flatk == i

        def bext(plane):
            return _rmax(jnp.where(sloti, plane, -2e9))

        x1i = bext(vx1)
        y1i = bext(vy1)
        x2i = bext(vx2)
        y2i = bext(vy2)
        ai = bext(var)
        si = bext(vsc)
        ki = bext(kv)
        xx1 = jnp.maximum(x1i, vx1)
        yy1 = jnp.maximum(y1i, vy1)
        xx2 = jnp.minimum(x2i, vx2)
        yy2 = jnp.minimum(y2i, vy2)
        inter = jnp.maximum(xx2 - xx1, 0.0) * jnp.maximum(yy2 - yy1, 0.0)
        iou = inter / (ai + var - inter + 1e-9)
        sup = ((iou > _NMS_T) & (flatk > i)
               & (ki > 0.5) & (si > 0.0))
        return jnp.where(sup, 0.0, kv)

    kvf = jax.lax.fori_loop(0, _K, nms_body,
                            jnp.ones((8, 128), jnp.float32), unroll=4)

    validk = flatk < _K
    fin0 = jnp.where(validk,
                     jnp.where((kvf > 0.5) & (vsc > 0.0), vsc, -1.0),
                     -1e9)

    def top_body(d, a):
        m11 = _rmax(a)
        j11 = _rmin(jnp.where(a == m11, flatk, _BIG))
        oh = flatk == j11

        def bext(plane):
            return _rmax(jnp.where(oh, plane, -2e9))

        b8 = lambda v: jnp.broadcast_to(v, (8, 128))
        slotd = flatk == d
        ox1[...] = jnp.where(slotd, b8(bext(vx1)), ox1[...])
        oy1[...] = jnp.where(slotd, b8(bext(vy1)), oy1[...])
        ox2[...] = jnp.where(slotd, b8(bext(vx2)), ox2[...])
        oy2[...] = jnp.where(slotd, b8(bext(vy2)), oy2[...])
        osc[...] = jnp.where(slotd, b8(m11), osc[...])
        return jnp.where(oh, -2e9, a)

    jax.lax.fori_loop(0, _D, top_body, fin0, unroll=4)


def _pad_plane(v, pad_val):
    p = jnp.concatenate(
        [v, jnp.full((_PADN - _N,), pad_val, v.dtype)])
    return p.reshape(_ROWS, 128)


def kernel(boxes, scores):
    boxes = boxes.astype(jnp.float32)
    scores = scores.astype(jnp.float32)
    bx1 = _pad_plane(boxes[:, 0], 0.0)
    by1 = _pad_plane(boxes[:, 1], 0.0)
    bx2 = _pad_plane(boxes[:, 2], 0.0)
    by2 = _pad_plane(boxes[:, 3], 0.0)
    sc = _pad_plane(scores, 0.0)
    shp = jax.ShapeDtypeStruct((8, 128), jnp.float32)
    ox1, oy1, ox2, oy2, osc = pl.pallas_call(
        _krn,
        out_shape=(shp, shp, shp, shp, shp),
        scratch_shapes=[
            pltpu.VMEM((8, 128), jnp.int32),
            pltpu.VMEM((8, 128), jnp.float32),
            pltpu.VMEM((8, 128), jnp.float32),
            pltpu.VMEM((8, 128), jnp.float32),
            pltpu.VMEM((8, 128), jnp.float32),
            pltpu.VMEM((8, 128), jnp.float32),
        ],
    )(bx1, by1, bx2, by2, sc)
    det = jnp.stack([ox1.reshape(-1)[:_D], oy1.reshape(-1)[:_D],
                     ox2.reshape(-1)[:_D], oy2.reshape(-1)[:_D],
                     osc.reshape(-1)[:_D]], axis=1)
    return det
